# baseline (speedup 1.0000x reference)
import jax
import jax.numpy as jnp
from jax import lax
from jax.experimental import pallas as pl
from jax.experimental.pallas import tpu as pltpu
from jax.experimental.pallas import tpu_sc as plsc

_BANK = 16
_BATCH = 32
_PAIRS = _BATCH * 2
_ROWS = 128
_HEAD_BLOCKS = 2


def _build_w_body(idx_hbm, prob_hbm, w_hbm, idx_v, prob_v, w_v):
    wid = lax.axis_index("s") * 2 + lax.axis_index("c")

    @pl.when(wid == 0)
    def _():
        pltpu.sync_copy(idx_hbm, idx_v)
        pltpu.sync_copy(prob_hbm, prob_v)
        for e in range(_BANK):
            for h in range(_PAIRS // 16):
                s = pl.ds(e * _BATCH + (h % 2) * 16, 16)
                idxc = idx_v[pl.ds(h * 16, 16)]
                probc = prob_v[pl.ds(h * 16, 16)]
                contrib = jnp.where(idxc == e, probc, jnp.zeros((16,), jnp.float32))
                if h < 2:
                    w_v[s] = contrib
                else:
                    w_v[s] = w_v[s] + contrib
        pltpu.sync_copy(w_v, w_hbm)


def _build_wt(selection_index, selection_probabilities):
    idx_flat = jnp.transpose(selection_index, (1, 0)).reshape(_PAIRS)
    prob_flat = jnp.transpose(selection_probabilities, (1, 0)).reshape(_PAIRS)
    mesh = plsc.VectorSubcoreMesh(core_axis_name="c", subcore_axis_name="s")
    wt = pl.kernel(
        _build_w_body,
        mesh=mesh,
        out_type=jax.ShapeDtypeStruct((_BANK * _BATCH,), jnp.float32),
        scratch_types=[
            pltpu.VMEM((_PAIRS,), jnp.int32),
            pltpu.VMEM((_PAIRS,), jnp.float32),
            pltpu.VMEM((_BANK * _BATCH,), jnp.float32),
        ],
    )(idx_flat.astype(jnp.int32), prob_flat)
    return wt.reshape(_BANK, _BATCH)


def _combine_head_body(idx_ref, prob_ref, p_ref, o_ref):
    idx = idx_ref[...]
    prob = prob_ref[...]
    e = jax.lax.broadcasted_iota(jnp.int32, (1, 1, _BANK), 2)
    onehot = (idx[:, :, None] == e).astype(jnp.float32)
    w = jnp.sum(prob[:, :, None] * onehot, axis=1)
    for r in range(_ROWS):
        o_ref[:, r, :] = jax.lax.dot_general(
            w, p_ref[r], (((1,), (0,)), ((), ())),
            preferred_element_type=jnp.float32)


def _combine_tail_body(wt_ref, p_ref, prev_ref, o_ref):
    del prev_ref
    wt = wt_ref[...]
    for r in range(_ROWS):
        o_ref[:, r, :] = jax.lax.dot_general(
            wt, p_ref[r], (((0,), (0,)), ((), ())),
            preferred_element_type=jnp.float32)


def kernel(parameter, selection_index, selection_probabilities):
    h, w_dim, bank = parameter.shape
    n_blocks = h // _ROWS
    wt = _build_wt(selection_index, selection_probabilities)
    p_t = jnp.transpose(parameter, (0, 2, 1))
    out_shape = jax.ShapeDtypeStruct((_BATCH, h, w_dim), jnp.float32)

    head = pl.pallas_call(
        _combine_head_body,
        grid=(_HEAD_BLOCKS,),
        in_specs=[
            pl.BlockSpec((_BATCH, 2), lambda i: (0, 0)),
            pl.BlockSpec((_BATCH, 2), lambda i: (0, 0)),
            pl.BlockSpec((_ROWS, bank, w_dim), lambda i: (i, 0, 0)),
        ],
        out_specs=pl.BlockSpec((_BATCH, _ROWS, w_dim), lambda i: (0, i, 0)),
        out_shape=out_shape,
    )(selection_index, selection_probabilities, p_t)

    out = pl.pallas_call(
        _combine_tail_body,
        grid=(n_blocks - _HEAD_BLOCKS,),
        in_specs=[
            pl.BlockSpec((_BANK, _BATCH), lambda i: (0, 0)),
            pl.BlockSpec(
                (_ROWS, bank, w_dim), lambda i: (i + _HEAD_BLOCKS, 0, 0)),
            pl.BlockSpec(memory_space=pl.ANY),
        ],
        out_specs=pl.BlockSpec(
            (_BATCH, _ROWS, w_dim), lambda i: (0, i + _HEAD_BLOCKS, 0)),
        out_shape=out_shape,
        input_output_aliases={2: 0},
    )(wt, p_t, head)
    return out


# speedup vs baseline: 1.0796x; 1.0796x over previous
import jax
import jax.numpy as jnp
from jax import lax
from jax.experimental import pallas as pl
from jax.experimental.pallas import tpu as pltpu
from jax.experimental.pallas import tpu_sc as plsc

_BANK = 16
_BATCH = 32
_PAIRS = _BATCH * 2
_ROWS = 128


def _build_w_body(idx_hbm, prob_hbm, w_hbm, idx_v, prob_v, w_v):
    wid = lax.axis_index("s") * 2 + lax.axis_index("c")

    @pl.when(wid == 0)
    def _():
        pltpu.sync_copy(idx_hbm, idx_v)
        pltpu.sync_copy(prob_hbm, prob_v)
        for e in range(_BANK):
            for h in range(_PAIRS // 16):
                s = pl.ds(e * _BATCH + (h % 2) * 16, 16)
                idxc = idx_v[pl.ds(h * 16, 16)]
                probc = prob_v[pl.ds(h * 16, 16)]
                contrib = jnp.where(idxc == e, probc, jnp.zeros((16,), jnp.float32))
                if h < 2:
                    w_v[s] = contrib
                else:
                    w_v[s] = w_v[s] + contrib
        pltpu.sync_copy(w_v, w_hbm)


def _build_wt(selection_index, selection_probabilities):
    idx_flat = jnp.transpose(selection_index, (1, 0)).reshape(_PAIRS)
    prob_flat = jnp.transpose(selection_probabilities, (1, 0)).reshape(_PAIRS)
    mesh = plsc.VectorSubcoreMesh(core_axis_name="c", subcore_axis_name="s")
    wt = pl.kernel(
        _build_w_body,
        mesh=mesh,
        out_type=jax.ShapeDtypeStruct((_BANK * _BATCH,), jnp.float32),
        scratch_types=[
            pltpu.VMEM((_PAIRS,), jnp.int32),
            pltpu.VMEM((_PAIRS,), jnp.float32),
            pltpu.VMEM((_BANK * _BATCH,), jnp.float32),
        ],
    )(idx_flat.astype(jnp.int32), prob_flat)
    return wt.reshape(_BANK, _BATCH)


def _combine_body(wt_ref, p_ref, o_ref):
    wt = wt_ref[...]
    for r in range(_ROWS):
        o_ref[:, r, :] = jax.lax.dot_general(
            wt, p_ref[r], (((0,), (0,)), ((), ())),
            preferred_element_type=jnp.float32)


def kernel(parameter, selection_index, selection_probabilities):
    h, w_dim, bank = parameter.shape
    wt = _build_wt(selection_index, selection_probabilities)
    p_t = jnp.transpose(parameter, (0, 2, 1))
    out = pl.pallas_call(
        _combine_body,
        grid=(h // _ROWS,),
        in_specs=[
            pl.BlockSpec((_BANK, _BATCH), lambda i: (0, 0)),
            pl.BlockSpec((_ROWS, bank, w_dim), lambda i: (i, 0, 0)),
        ],
        out_specs=pl.BlockSpec((_BATCH, _ROWS, w_dim), lambda i: (0, i, 0)),
        out_shape=jax.ShapeDtypeStruct((_BATCH, h, w_dim), jnp.float32),
    )(wt, p_t)
    return out
